# Initial kernel scaffold; baseline (speedup 1.0000x reference)
import jax
import jax.numpy as jnp
from jax.experimental import pallas as pl


def kernel(x, table):
    raise NotImplementedError("write your pallas kernel here")


# speedup vs baseline: 1.0244x; 1.0244x over previous
import functools

import jax
import jax.numpy as jnp
from jax import lax
from jax.experimental import pallas as pl
from jax.experimental.pallas import tpu as pltpu
from jax.experimental.pallas import tpu_sc as plsc

NC = 2
NS = 16
NW = NC * NS
CH = 128


def _gather_body(table_hbm, idx_hbm, out_hbm, idx_v, rows_v, sem, *, chunks, emb):
    wid = lax.axis_index("s") * NC + lax.axis_index("c")
    base = wid * chunks
    pltpu.sync_copy(idx_hbm.at[pl.ds(base, chunks)], idx_v)

    def body(j, carry):
        pltpu.async_copy(table_hbm.at[idx_v.at[j]], rows_v, sem).wait()
        pltpu.sync_copy(rows_v, out_hbm.at[pl.ds((base + j) * CH, CH)])
        return carry

    lax.fori_loop(0, chunks, body, 0)


def kernel(x, table):
    b, h = x.shape
    vocab, emb = table.shape
    total = b * h
    assert total % (NW * CH) == 0
    chunks = total // (NW * CH)

    idx = x.reshape(NW * chunks, CH).astype(jnp.int32)

    mesh = plsc.VectorSubcoreMesh(core_axis_name="c", subcore_axis_name="s")
    k = pl.kernel(
        functools.partial(_gather_body, chunks=chunks, emb=emb),
        out_type=jax.ShapeDtypeStruct((total, emb), jnp.float32),
        mesh=mesh,
        scratch_types=[
            pltpu.VMEM((chunks, CH), jnp.int32),
            pltpu.VMEM((CH, emb), jnp.float32),
            pltpu.SemaphoreType.DMA,
        ],
        compiler_params=pltpu.CompilerParams(use_tc_tiling_on_sc=False),
    )
    out = k(table, idx)
    return out.reshape(b, h, emb)


# speedup vs baseline: 1.1098x; 1.0834x over previous
import functools

import jax
import jax.numpy as jnp
from jax import lax
from jax.experimental import pallas as pl
from jax.experimental.pallas import tpu as pltpu
from jax.experimental.pallas import tpu_sc as plsc

NC = 2
NS = 16
NW = NC * NS
CH = 128
K = 10


def _gather_body(table_hbm, idx_hbm, out_hbm, idx_v, rows_v, gsA, gsB, osA,
                 osB, *, chunks, emb):
    wid = lax.axis_index("s") * NC + lax.axis_index("c")
    base = wid * chunks
    rounds = chunks // (2 * K)

    pltpu.sync_copy(idx_hbm.at[pl.ds(base, chunks)], idx_v)

    def fire_gather(c, slot, sem):
        pltpu.async_copy(table_hbm.at[idx_v.at[c]], rows_v.at[slot], sem)

    def drain_gather(slot, sem):
        pltpu.make_async_copy(table_hbm.at[pl.ds(0, CH)], rows_v.at[slot],
                              sem).wait()

    def fire_out(c, slot, sem):
        pltpu.async_copy(rows_v.at[slot], out_hbm.at[pl.ds((base + c) * CH, CH)],
                         sem)

    def drain_out(slot, sem):
        pltpu.make_async_copy(table_hbm.at[pl.ds(0, CH)],
                              rows_v.at[slot], sem).wait()

    def round_body(t, *, first, last):
        cA = 2 * K * t
        cB = cA + K
        for b in range(K):
            drain_gather(b, gsA)
        if not first:
            for b in range(K):
                drain_out(K + b, osB)
        for b in range(K):
            fire_gather(cB + b, K + b, gsB)
        for b in range(K):
            fire_out(cA + b, b, osA)
        for b in range(K):
            drain_gather(K + b, gsB)
        for b in range(K):
            drain_out(b, osA)
        if not last:
            for b in range(K):
                fire_gather(cA + 2 * K + b, b, gsA)
        for b in range(K):
            fire_out(cB + b, K + b, osB)

    for b in range(K):
        fire_gather(b, b, gsA)
    round_body(0, first=True, last=(rounds == 1))

    def mid(t, carry):
        round_body(t, first=False, last=False)
        return carry

    if rounds > 2:
        lax.fori_loop(1, rounds - 1, mid, 0)
    if rounds > 1:
        round_body(rounds - 1, first=False, last=True)

    for b in range(K):
        drain_out(K + b, osB)


def kernel(x, table):
    b, h = x.shape
    vocab, emb = table.shape
    total = b * h
    assert total % (NW * CH) == 0
    chunks = total // (NW * CH)
    assert chunks % (2 * K) == 0

    idx = x.reshape(NW * chunks, CH).astype(jnp.int32)

    mesh = plsc.VectorSubcoreMesh(core_axis_name="c", subcore_axis_name="s")
    k = pl.kernel(
        functools.partial(_gather_body, chunks=chunks, emb=emb),
        out_type=jax.ShapeDtypeStruct((total, emb), jnp.float32),
        mesh=mesh,
        scratch_types=[
            pltpu.VMEM((chunks, CH), jnp.int32),
            pltpu.VMEM((2 * K, CH, emb), jnp.float32),
            pltpu.SemaphoreType.DMA,
            pltpu.SemaphoreType.DMA,
            pltpu.SemaphoreType.DMA,
            pltpu.SemaphoreType.DMA,
        ],
        compiler_params=pltpu.CompilerParams(use_tc_tiling_on_sc=False),
    )
    out = k(table, idx)
    return out.reshape(b, h, emb)


# speedup vs baseline: 1.7887x; 1.6117x over previous
import functools

import jax
import jax.numpy as jnp
from jax import lax
from jax.experimental import pallas as pl
from jax.experimental.pallas import tpu as pltpu
from jax.experimental.pallas import tpu_sc as plsc

NC = 2
NS = 16
NW = NC * NS
K = 16


def _gather_body(table_hbm, x_hbm, out_hbm, idx_v, rows_v, gsA, gsB, osA,
                 osB, *, nb, hist, emb):
    wid = lax.axis_index("s") * NC + lax.axis_index("c")
    base = wid * nb
    rounds = nb // (2 * K)

    pltpu.sync_copy(x_hbm.at[pl.ds(base, nb)], idx_v)

    def fire_gather(s, slot, sem):
        pltpu.async_copy(table_hbm.at[idx_v.at[s]], rows_v.at[slot], sem)

    def drain_gather(slot, sem):
        pltpu.make_async_copy(out_hbm.at[0], rows_v.at[slot], sem).wait()

    def fire_out(s, slot, sem):
        pltpu.async_copy(rows_v.at[slot], out_hbm.at[base + s], sem)

    def drain_out(slot, sem):
        pltpu.make_async_copy(out_hbm.at[0], rows_v.at[slot], sem).wait()

    def round_body(t, *, first, last):
        sA = 2 * K * t
        sB = sA + K
        for b in range(K):
            drain_gather(b, gsA)
        if not first:
            for b in range(K):
                drain_out(K + b, osB)
        for b in range(K):
            fire_gather(sB + b, K + b, gsB)
        for b in range(K):
            fire_out(sA + b, b, osA)
        for b in range(K):
            drain_gather(K + b, gsB)
        for b in range(K):
            drain_out(b, osA)
        if not last:
            for b in range(K):
                fire_gather(sA + 2 * K + b, b, gsA)
        for b in range(K):
            fire_out(sB + b, K + b, osB)

    for b in range(K):
        fire_gather(b, b, gsA)
    round_body(0, first=True, last=(rounds == 1))

    def mid(t, carry):
        round_body(t, first=False, last=False)
        return carry

    if rounds > 2:
        lax.fori_loop(1, rounds - 1, mid, 0)
    if rounds > 1:
        round_body(rounds - 1, first=False, last=True)

    for b in range(K):
        drain_out(K + b, osB)


def kernel(x, table):
    bsz, hist = x.shape
    vocab, emb = table.shape
    assert bsz % NW == 0
    nb = bsz // NW
    assert nb % (2 * K) == 0

    mesh = plsc.VectorSubcoreMesh(core_axis_name="c", subcore_axis_name="s")
    k = pl.kernel(
        functools.partial(_gather_body, nb=nb, hist=hist, emb=emb),
        out_type=jax.ShapeDtypeStruct((bsz, hist, emb), jnp.float32),
        mesh=mesh,
        scratch_types=[
            pltpu.VMEM((nb, hist), jnp.int32),
            pltpu.VMEM((2 * K, hist, emb), jnp.float32),
            pltpu.SemaphoreType.DMA,
            pltpu.SemaphoreType.DMA,
            pltpu.SemaphoreType.DMA,
            pltpu.SemaphoreType.DMA,
        ],
        compiler_params=pltpu.CompilerParams(use_tc_tiling_on_sc=False),
    )
    return k(table, x.astype(jnp.int32))
